# initial kernel scaffold (unmeasured)
import jax
import jax.numpy as jnp
from jax import lax
from jax.experimental import pallas as pl
from jax.experimental.pallas import tpu as pltpu

N_DEV = 4


def kernel(A, B):
    m, k = A.shape
    _, n = B.shape
    m_chunk = m // N_DEV

    out_dtype = jnp.bfloat16

    def body(a_ref, b_ref, out_ref, acc_ref, send_buf, rs_recv,
             rs_send_sems, rs_recv_sems, ag_send_sems, ag_recv_sems):
        my = lax.axis_index("i")
        left = lax.rem(my + N_DEV - 1, N_DEV)
        right = lax.rem(my + 1, N_DEV)

        barrier_sem = pltpu.get_barrier_semaphore()
        for nbr in [left, right]:
            pl.semaphore_signal(
                barrier_sem, inc=1,
                device_id=(nbr,), device_id_type=pl.DeviceIdType.MESH,
            )
        pl.semaphore_wait(barrier_sem, 2)

        a = a_ref[...].astype(jnp.bfloat16)
        b = b_ref[...].astype(jnp.bfloat16)
        acc_ref[...] = jnp.dot(a, b, preferred_element_type=jnp.float32)

        for s in range(N_DEV - 1):
            send_chunk = lax.rem(my + N_DEV - 1 - s, N_DEV)
            recv_chunk = lax.rem(my + N_DEV - 2 - s, N_DEV)
            if s == 0:
                send_buf[0] = acc_ref[
                    pl.ds(send_chunk * m_chunk, m_chunk), :
                ].astype(jnp.bfloat16)
            rdma = pltpu.make_async_remote_copy(
                src_ref=send_buf.at[s],
                dst_ref=rs_recv.at[s],
                send_sem=rs_send_sems.at[s],
                recv_sem=rs_recv_sems.at[s],
                device_id=(right,),
                device_id_type=pl.DeviceIdType.MESH,
            )
            rdma.start()
            rdma.wait()
            acc_chunk = (
                acc_ref[pl.ds(recv_chunk * m_chunk, m_chunk), :]
                + rs_recv[s].astype(jnp.float32)
            )
            if s < N_DEV - 2:
                send_buf[s + 1] = acc_chunk.astype(jnp.bfloat16)
            else:
                out_ref[pl.ds(my * m_chunk, m_chunk), :] = jnp.maximum(
                    acc_chunk, 0.0
                ).astype(out_dtype)

        for t in range(N_DEV - 1):
            send_chunk = lax.rem(my + N_DEV - t, N_DEV)
            rdma = pltpu.make_async_remote_copy(
                src_ref=out_ref.at[pl.ds(send_chunk * m_chunk, m_chunk), :],
                dst_ref=out_ref.at[pl.ds(send_chunk * m_chunk, m_chunk), :],
                send_sem=ag_send_sems.at[t],
                recv_sem=ag_recv_sems.at[t],
                device_id=(right,),
                device_id_type=pl.DeviceIdType.MESH,
            )
            rdma.start()
            rdma.wait()

    return pl.pallas_call(
        body,
        out_shape=jax.ShapeDtypeStruct((m, n), out_dtype),
        in_specs=[
            pl.BlockSpec(memory_space=pltpu.VMEM),
            pl.BlockSpec(memory_space=pltpu.VMEM),
        ],
        out_specs=pl.BlockSpec(memory_space=pltpu.VMEM),
        scratch_shapes=[
            pltpu.VMEM((m, n), jnp.float32),
            pltpu.VMEM((N_DEV - 1, m_chunk, n), jnp.bfloat16),
            pltpu.VMEM((N_DEV - 1, m_chunk, n), jnp.bfloat16),
            pltpu.SemaphoreType.DMA((N_DEV - 1,)),
            pltpu.SemaphoreType.DMA((N_DEV - 1,)),
            pltpu.SemaphoreType.DMA((N_DEV - 1,)),
            pltpu.SemaphoreType.DMA((N_DEV - 1,)),
        ],
        compiler_params=pltpu.CompilerParams(collective_id=0),
    )(A, B)


# baseline (device time: 175586 ns/iter reference)
import jax
import jax.numpy as jnp
from jax import lax
from jax.experimental import pallas as pl
from jax.experimental.pallas import tpu as pltpu

N_DEV = 4


def kernel(A, B):
    m, k = A.shape
    _, n = B.shape
    m_chunk = m // N_DEV

    out_dtype = jnp.bfloat16

    def body(a_ref, b_ref, out_ref, acc_ref, send_buf, rs_recv,
             rs_send_sems, rs_recv_sems, ag_send_sems, ag_recv_sems):
        my = lax.axis_index("i")
        left = lax.rem(my + N_DEV - 1, N_DEV)
        right = lax.rem(my + 1, N_DEV)

        barrier_sem = pltpu.get_barrier_semaphore()
        for nbr in [left, right]:
            pl.semaphore_signal(
                barrier_sem, inc=1,
                device_id=(nbr,), device_id_type=pl.DeviceIdType.MESH,
            )
        pl.semaphore_wait(barrier_sem, 2)

        a = a_ref[...].astype(jnp.bfloat16)
        b = b_ref[...].astype(jnp.bfloat16)
        acc_ref[...] = jnp.dot(a, b, preferred_element_type=jnp.float32)

        for s in range(N_DEV - 1):
            send_chunk = lax.rem(my + N_DEV - 1 - s, N_DEV)
            recv_chunk = lax.rem(my + N_DEV - 2 - s, N_DEV)
            if s == 0:
                send_buf[0] = acc_ref[
                    pl.ds(send_chunk * m_chunk, m_chunk), :
                ].astype(jnp.bfloat16)
            rdma = pltpu.make_async_remote_copy(
                src_ref=send_buf.at[s],
                dst_ref=rs_recv.at[s],
                send_sem=rs_send_sems.at[s],
                recv_sem=rs_recv_sems.at[s],
                device_id=(right,),
                device_id_type=pl.DeviceIdType.MESH,
            )
            rdma.start()
            rdma.wait()
            acc_chunk = (
                acc_ref[pl.ds(recv_chunk * m_chunk, m_chunk), :]
                + rs_recv[s].astype(jnp.float32)
            )
            if s < N_DEV - 2:
                send_buf[s + 1] = acc_chunk.astype(jnp.bfloat16)
            else:
                out_ref[pl.ds(my * m_chunk, m_chunk), :] = jnp.maximum(
                    acc_chunk, 0.0
                ).astype(out_dtype)

        for t in range(N_DEV - 1):
            send_chunk = lax.rem(my + N_DEV - t, N_DEV)
            rdma = pltpu.make_async_remote_copy(
                src_ref=out_ref.at[pl.ds(send_chunk * m_chunk, m_chunk), :],
                dst_ref=out_ref.at[pl.ds(send_chunk * m_chunk, m_chunk), :],
                send_sem=ag_send_sems.at[t],
                recv_sem=ag_recv_sems.at[t],
                device_id=(right,),
                device_id_type=pl.DeviceIdType.MESH,
            )
            rdma.start()
            rdma.wait()

    return pl.pallas_call(
        body,
        out_shape=jax.ShapeDtypeStruct((m, n), out_dtype),
        in_specs=[
            pl.BlockSpec(memory_space=pltpu.VMEM),
            pl.BlockSpec(memory_space=pltpu.VMEM),
        ],
        out_specs=pl.BlockSpec(memory_space=pltpu.VMEM),
        scratch_shapes=[
            pltpu.VMEM((m, n), jnp.float32),
            pltpu.VMEM((N_DEV - 1, m_chunk, n), jnp.bfloat16),
            pltpu.VMEM((N_DEV - 1, m_chunk, n), jnp.bfloat16),
            pltpu.SemaphoreType.DMA((N_DEV - 1,)),
            pltpu.SemaphoreType.DMA((N_DEV - 1,)),
            pltpu.SemaphoreType.DMA((N_DEV - 1,)),
            pltpu.SemaphoreType.DMA((N_DEV - 1,)),
        ],
        compiler_params=pltpu.CompilerParams(
            collective_id=0,
            vmem_limit_bytes=100 * 1024 * 1024,
        ),
    )(A, B)


# device time: 108412 ns/iter; 1.6196x vs baseline; 1.6196x over previous
import jax
import jax.numpy as jnp
from jax import lax
from jax.experimental import pallas as pl
from jax.experimental.pallas import tpu as pltpu

N_DEV = 4


def kernel(A, B):
    m, k = A.shape
    _, n = B.shape
    half = m // 2
    mc = half // N_DEV

    out_dtype = jnp.bfloat16

    def body(a_ref, b_ref, out_ref, acc_ref,
             send_r, send_l, recv_r, recv_l,
             r_send_sems, r_recv_sems, l_send_sems, l_recv_sems,
             agr_send_sems, agr_recv_sems, agl_send_sems, agl_recv_sems):
        my = lax.axis_index("i")
        left = lax.rem(my + N_DEV - 1, N_DEV)
        right = lax.rem(my + 1, N_DEV)

        barrier_sem = pltpu.get_barrier_semaphore()
        for nbr in [left, right]:
            pl.semaphore_signal(
                barrier_sem, inc=1,
                device_id=(nbr,), device_id_type=pl.DeviceIdType.MESH,
            )
        pl.semaphore_wait(barrier_sem, 2)

        a = a_ref[...].astype(jnp.bfloat16)
        b = b_ref[...].astype(jnp.bfloat16)
        acc_ref[...] = jnp.dot(a, b, preferred_element_type=jnp.float32)

        def top_rows(c):
            return pl.ds(c * mc, mc)

        def bot_rows(c):
            return pl.ds(half + c * mc, mc)

        for s in range(N_DEV - 1):
            sc_r = lax.rem(my + N_DEV - 1 - s, N_DEV)
            rc_r = lax.rem(my + N_DEV - 2 - s, N_DEV)
            sc_l = lax.rem(my + 1 + s, N_DEV)
            rc_l = lax.rem(my + 2 + s, N_DEV)
            if s == 0:
                send_r[0] = acc_ref[top_rows(sc_r), :].astype(jnp.bfloat16)
                send_l[0] = acc_ref[bot_rows(sc_l), :].astype(jnp.bfloat16)
            rdma_r = pltpu.make_async_remote_copy(
                src_ref=send_r.at[s], dst_ref=recv_r.at[s],
                send_sem=r_send_sems.at[s], recv_sem=r_recv_sems.at[s],
                device_id=(right,), device_id_type=pl.DeviceIdType.MESH,
            )
            rdma_l = pltpu.make_async_remote_copy(
                src_ref=send_l.at[s], dst_ref=recv_l.at[s],
                send_sem=l_send_sems.at[s], recv_sem=l_recv_sems.at[s],
                device_id=(left,), device_id_type=pl.DeviceIdType.MESH,
            )
            rdma_r.start()
            rdma_l.start()
            rdma_r.wait()
            rdma_l.wait()
            acc_r = acc_ref[top_rows(rc_r), :] + recv_r[s].astype(jnp.float32)
            acc_l = acc_ref[bot_rows(rc_l), :] + recv_l[s].astype(jnp.float32)
            if s < N_DEV - 2:
                send_r[s + 1] = acc_r.astype(jnp.bfloat16)
                send_l[s + 1] = acc_l.astype(jnp.bfloat16)
            else:
                out_ref[top_rows(my), :] = jnp.maximum(acc_r, 0.0).astype(out_dtype)
                out_ref[bot_rows(my), :] = jnp.maximum(acc_l, 0.0).astype(out_dtype)

        for t in range(N_DEV - 1):
            sc_r = lax.rem(my + N_DEV - t, N_DEV)
            sc_l = lax.rem(my + t, N_DEV)
            rdma_r = pltpu.make_async_remote_copy(
                src_ref=out_ref.at[top_rows(sc_r), :],
                dst_ref=out_ref.at[top_rows(sc_r), :],
                send_sem=agr_send_sems.at[t], recv_sem=agr_recv_sems.at[t],
                device_id=(right,), device_id_type=pl.DeviceIdType.MESH,
            )
            rdma_l = pltpu.make_async_remote_copy(
                src_ref=out_ref.at[bot_rows(sc_l), :],
                dst_ref=out_ref.at[bot_rows(sc_l), :],
                send_sem=agl_send_sems.at[t], recv_sem=agl_recv_sems.at[t],
                device_id=(left,), device_id_type=pl.DeviceIdType.MESH,
            )
            rdma_r.start()
            rdma_l.start()
            rdma_r.wait()
            rdma_l.wait()

    return pl.pallas_call(
        body,
        out_shape=jax.ShapeDtypeStruct((m, n), out_dtype),
        in_specs=[
            pl.BlockSpec(memory_space=pltpu.VMEM),
            pl.BlockSpec(memory_space=pltpu.VMEM),
        ],
        out_specs=pl.BlockSpec(memory_space=pltpu.VMEM),
        scratch_shapes=[
            pltpu.VMEM((m, n), jnp.float32),
            pltpu.VMEM((N_DEV - 1, mc, n), jnp.bfloat16),
            pltpu.VMEM((N_DEV - 1, mc, n), jnp.bfloat16),
            pltpu.VMEM((N_DEV - 1, mc, n), jnp.bfloat16),
            pltpu.VMEM((N_DEV - 1, mc, n), jnp.bfloat16),
            pltpu.SemaphoreType.DMA((N_DEV - 1,)),
            pltpu.SemaphoreType.DMA((N_DEV - 1,)),
            pltpu.SemaphoreType.DMA((N_DEV - 1,)),
            pltpu.SemaphoreType.DMA((N_DEV - 1,)),
            pltpu.SemaphoreType.DMA((N_DEV - 1,)),
            pltpu.SemaphoreType.DMA((N_DEV - 1,)),
            pltpu.SemaphoreType.DMA((N_DEV - 1,)),
            pltpu.SemaphoreType.DMA((N_DEV - 1,)),
        ],
        compiler_params=pltpu.CompilerParams(
            collective_id=0,
            vmem_limit_bytes=100 * 1024 * 1024,
        ),
    )(A, B)


# device time: 102113 ns/iter; 1.7195x vs baseline; 1.0617x over previous
import jax
import jax.numpy as jnp
from jax import lax
from jax.experimental import pallas as pl
from jax.experimental.pallas import tpu as pltpu

N_DEV = 4


def kernel(A, B):
    m, k = A.shape
    _, n = B.shape
    half = m // 2
    mc = half // N_DEV

    out_dtype = jnp.bfloat16
    bf16 = jnp.bfloat16
    f32 = jnp.float32

    def body(a_ref, b_ref, out_ref, a_bf, b_bf, acc_ref,
             send_r, send_l, recv_r, recv_l,
             r_send_sems, r_recv_sems, l_send_sems, l_recv_sems,
             agr_send_sems, agr_recv_sems, agl_send_sems, agl_recv_sems):
        my = lax.axis_index("i")
        left = lax.rem(my + N_DEV - 1, N_DEV)
        right = lax.rem(my + 1, N_DEV)

        barrier_sem = pltpu.get_barrier_semaphore()
        for nbr in [left, right]:
            pl.semaphore_signal(
                barrier_sem, inc=1,
                device_id=(nbr,), device_id_type=pl.DeviceIdType.MESH,
            )
        pl.semaphore_wait(barrier_sem, 2)

        a_bf[...] = a_ref[...].astype(bf16)
        b_bf[...] = b_ref[...].astype(bf16)

        def top_rows(c):
            return pl.ds(c * mc, mc)

        def bot_rows(c):
            return pl.ds(half + c * mc, mc)

        def dot_rows(rows):
            return jnp.dot(a_bf[rows, :], b_bf[...], preferred_element_type=f32)

        def mk_rs(s, sr, rr, ss, rs, dev):
            return pltpu.make_async_remote_copy(
                src_ref=sr.at[s], dst_ref=rr.at[s],
                send_sem=ss.at[s], recv_sem=rs.at[s],
                device_id=(dev,), device_id_type=pl.DeviceIdType.MESH,
            )


        send_r[0] = dot_rows(top_rows(lax.rem(my + N_DEV - 1, N_DEV))).astype(bf16)
        send_l[0] = dot_rows(bot_rows(lax.rem(my + 1, N_DEV))).astype(bf16)
        rdma_r = mk_rs(0, send_r, recv_r, r_send_sems, r_recv_sems, right)
        rdma_l = mk_rs(0, send_l, recv_l, l_send_sems, l_recv_sems, left)
        rdma_r.start()
        rdma_l.start()

        for s in range(N_DEV - 1):
            rc_r = lax.rem(my + N_DEV - 2 - s, N_DEV)
            rc_l = lax.rem(my + 2 + s, N_DEV)
            acc_ref[top_rows(rc_r), :] = dot_rows(top_rows(rc_r))
            acc_ref[bot_rows(rc_l), :] = dot_rows(bot_rows(rc_l))
            rdma_r.wait()
            rdma_l.wait()
            acc_r = acc_ref[top_rows(rc_r), :] + recv_r[s].astype(f32)
            acc_l = acc_ref[bot_rows(rc_l), :] + recv_l[s].astype(f32)
            if s < N_DEV - 2:
                send_r[s + 1] = acc_r.astype(bf16)
                send_l[s + 1] = acc_l.astype(bf16)
                rdma_r = mk_rs(s + 1, send_r, recv_r, r_send_sems, r_recv_sems, right)
                rdma_l = mk_rs(s + 1, send_l, recv_l, l_send_sems, l_recv_sems, left)
                rdma_r.start()
                rdma_l.start()
            else:
                out_ref[top_rows(my), :] = jnp.maximum(acc_r, 0.0).astype(out_dtype)
                out_ref[bot_rows(my), :] = jnp.maximum(acc_l, 0.0).astype(out_dtype)

        for t in range(N_DEV - 1):
            sc_r = lax.rem(my + N_DEV - t, N_DEV)
            sc_l = lax.rem(my + t, N_DEV)
            rdma_r = pltpu.make_async_remote_copy(
                src_ref=out_ref.at[top_rows(sc_r), :],
                dst_ref=out_ref.at[top_rows(sc_r), :],
                send_sem=agr_send_sems.at[t], recv_sem=agr_recv_sems.at[t],
                device_id=(right,), device_id_type=pl.DeviceIdType.MESH,
            )
            rdma_l = pltpu.make_async_remote_copy(
                src_ref=out_ref.at[bot_rows(sc_l), :],
                dst_ref=out_ref.at[bot_rows(sc_l), :],
                send_sem=agl_send_sems.at[t], recv_sem=agl_recv_sems.at[t],
                device_id=(left,), device_id_type=pl.DeviceIdType.MESH,
            )
            rdma_r.start()
            rdma_l.start()
            rdma_r.wait()
            rdma_l.wait()

    return pl.pallas_call(
        body,
        out_shape=jax.ShapeDtypeStruct((m, n), out_dtype),
        in_specs=[
            pl.BlockSpec(memory_space=pltpu.VMEM),
            pl.BlockSpec(memory_space=pltpu.VMEM),
        ],
        out_specs=pl.BlockSpec(memory_space=pltpu.VMEM),
        scratch_shapes=[
            pltpu.VMEM((m, k), bf16),
            pltpu.VMEM((k, n), bf16),
            pltpu.VMEM((m, n), f32),
            pltpu.VMEM((N_DEV - 1, mc, n), bf16),
            pltpu.VMEM((N_DEV - 1, mc, n), bf16),
            pltpu.VMEM((N_DEV - 1, mc, n), bf16),
            pltpu.VMEM((N_DEV - 1, mc, n), bf16),
            pltpu.SemaphoreType.DMA((N_DEV - 1,)),
            pltpu.SemaphoreType.DMA((N_DEV - 1,)),
            pltpu.SemaphoreType.DMA((N_DEV - 1,)),
            pltpu.SemaphoreType.DMA((N_DEV - 1,)),
            pltpu.SemaphoreType.DMA((N_DEV - 1,)),
            pltpu.SemaphoreType.DMA((N_DEV - 1,)),
            pltpu.SemaphoreType.DMA((N_DEV - 1,)),
            pltpu.SemaphoreType.DMA((N_DEV - 1,)),
        ],
        compiler_params=pltpu.CompilerParams(
            collective_id=0,
            vmem_limit_bytes=100 * 1024 * 1024,
        ),
    )(A, B)


# device time: 91664 ns/iter; 1.9155x vs baseline; 1.1140x over previous
import jax
import jax.numpy as jnp
from jax import lax
from jax.experimental import pallas as pl
from jax.experimental.pallas import tpu as pltpu

N_DEV = 4
SUB = 2


def kernel(A, B):
    m, k = A.shape
    _, n = B.shape
    half = m // 2
    mc = half // N_DEV
    msc = mc // SUB

    out_dtype = jnp.bfloat16
    bf16 = jnp.bfloat16
    f32 = jnp.float32

    def body(a_ref, b_ref, out_ref, b_bf, acc_ref,
             send_r, send_l, recv_r, recv_l,
             r_send_sems, r_recv_sems, l_send_sems, l_recv_sems,
             agr_send_sems, agr_recv_sems, agl_send_sems, agl_recv_sems):
        my = lax.axis_index("i")
        left = lax.rem(my + N_DEV - 1, N_DEV)
        right = lax.rem(my + 1, N_DEV)

        barrier_sem = pltpu.get_barrier_semaphore()
        for nbr in [left, right]:
            pl.semaphore_signal(
                barrier_sem, inc=1,
                device_id=(nbr,), device_id_type=pl.DeviceIdType.MESH,
            )
        pl.semaphore_wait(barrier_sem, 2)

        b_bf[...] = b_ref[...].astype(bf16)

        def top_rows(c):
            return pl.ds(c * mc, mc)

        def bot_rows(c):
            return pl.ds(half + c * mc, mc)

        def dot_rows(rows):
            return jnp.dot(
                a_ref[rows, :].astype(bf16), b_bf[...],
                preferred_element_type=f32,
            )

        def mk_rs(s, j, sbuf, rbuf, ssem, rsem, dev):
            sub = pl.ds(j * msc, msc)
            return pltpu.make_async_remote_copy(
                src_ref=sbuf.at[s, sub, :], dst_ref=rbuf.at[s, sub, :],
                send_sem=ssem.at[s, j], recv_sem=rsem.at[s, j],
                device_id=(dev,), device_id_type=pl.DeviceIdType.MESH,
            )

        def mk_ag(t, j, chunk_rows_fn, chunk, ssem, rsem, dev):
            sub = pl.ds(chunk * mc + j * msc, msc)
            rows = (
                sub if chunk_rows_fn is top_rows
                else pl.ds(half + chunk * mc + j * msc, msc)
            )
            return pltpu.make_async_remote_copy(
                src_ref=out_ref.at[rows, :], dst_ref=out_ref.at[rows, :],
                send_sem=ssem.at[t, j], recv_sem=rsem.at[t, j],
                device_id=(dev,), device_id_type=pl.DeviceIdType.MESH,
            )


        send_r[0] = dot_rows(top_rows(lax.rem(my + N_DEV - 1, N_DEV))).astype(bf16)
        send_l[0] = dot_rows(bot_rows(lax.rem(my + 1, N_DEV))).astype(bf16)
        rs_r = {}
        rs_l = {}
        for j in range(SUB):
            rs_r[(0, j)] = mk_rs(0, j, send_r, recv_r, r_send_sems, r_recv_sems, right)
            rs_l[(0, j)] = mk_rs(0, j, send_l, recv_l, l_send_sems, l_recv_sems, left)
            rs_r[(0, j)].start()
            rs_l[(0, j)].start()

        ag_r = {}
        ag_l = {}
        for s in range(N_DEV - 1):
            rc_r = lax.rem(my + N_DEV - 2 - s, N_DEV)
            rc_l = lax.rem(my + 2 + s, N_DEV)
            acc_ref[top_rows(rc_r), :] = dot_rows(top_rows(rc_r))
            acc_ref[bot_rows(rc_l), :] = dot_rows(bot_rows(rc_l))
            for j in range(SUB):
                sub = pl.ds(j * msc, msc)
                sub_top = pl.ds(rc_r * mc + j * msc, msc)
                sub_bot = pl.ds(half + rc_l * mc + j * msc, msc)
                rs_r[(s, j)].wait()
                acc_rj = acc_ref[sub_top, :] + recv_r[s, sub, :].astype(f32)
                rs_l[(s, j)].wait()
                acc_lj = acc_ref[sub_bot, :] + recv_l[s, sub, :].astype(f32)
                if s < N_DEV - 2:
                    send_r[s + 1, sub, :] = acc_rj.astype(bf16)
                    send_l[s + 1, sub, :] = acc_lj.astype(bf16)
                    rs_r[(s + 1, j)] = mk_rs(
                        s + 1, j, send_r, recv_r, r_send_sems, r_recv_sems, right)
                    rs_l[(s + 1, j)] = mk_rs(
                        s + 1, j, send_l, recv_l, l_send_sems, l_recv_sems, left)
                    rs_r[(s + 1, j)].start()
                    rs_l[(s + 1, j)].start()
                else:
                    out_ref[sub_top, :] = jnp.maximum(acc_rj, 0.0).astype(out_dtype)
                    out_ref[sub_bot, :] = jnp.maximum(acc_lj, 0.0).astype(out_dtype)
                    ag_r[(0, j)] = mk_ag(
                        0, j, top_rows, my, agr_send_sems, agr_recv_sems, right)
                    ag_l[(0, j)] = mk_ag(
                        0, j, bot_rows, my, agl_send_sems, agl_recv_sems, left)
                    ag_r[(0, j)].start()
                    ag_l[(0, j)].start()

        for t in range(N_DEV - 1):
            fc_r = lax.rem(my + N_DEV - 1 - t, N_DEV)
            fc_l = lax.rem(my + 1 + t, N_DEV)
            for j in range(SUB):
                ag_r[(t, j)].wait()
                ag_l[(t, j)].wait()
                if t < N_DEV - 2:
                    ag_r[(t + 1, j)] = mk_ag(
                        t + 1, j, top_rows, fc_r, agr_send_sems, agr_recv_sems, right)
                    ag_l[(t + 1, j)] = mk_ag(
                        t + 1, j, bot_rows, fc_l, agl_send_sems, agl_recv_sems, left)
                    ag_r[(t + 1, j)].start()
                    ag_l[(t + 1, j)].start()

    return pl.pallas_call(
        body,
        out_shape=jax.ShapeDtypeStruct((m, n), out_dtype),
        in_specs=[
            pl.BlockSpec(memory_space=pltpu.VMEM),
            pl.BlockSpec(memory_space=pltpu.VMEM),
        ],
        out_specs=pl.BlockSpec(memory_space=pltpu.VMEM),
        scratch_shapes=[
            pltpu.VMEM((k, n), bf16),
            pltpu.VMEM((m, n), f32),
            pltpu.VMEM((N_DEV - 1, mc, n), bf16),
            pltpu.VMEM((N_DEV - 1, mc, n), bf16),
            pltpu.VMEM((N_DEV - 1, mc, n), bf16),
            pltpu.VMEM((N_DEV - 1, mc, n), bf16),
            pltpu.SemaphoreType.DMA((N_DEV - 1, SUB)),
            pltpu.SemaphoreType.DMA((N_DEV - 1, SUB)),
            pltpu.SemaphoreType.DMA((N_DEV - 1, SUB)),
            pltpu.SemaphoreType.DMA((N_DEV - 1, SUB)),
            pltpu.SemaphoreType.DMA((N_DEV - 1, SUB)),
            pltpu.SemaphoreType.DMA((N_DEV - 1, SUB)),
            pltpu.SemaphoreType.DMA((N_DEV - 1, SUB)),
            pltpu.SemaphoreType.DMA((N_DEV - 1, SUB)),
        ],
        compiler_params=pltpu.CompilerParams(
            collective_id=0,
            vmem_limit_bytes=100 * 1024 * 1024,
        ),
    )(A, B)
